# Initial kernel scaffold; baseline (speedup 1.0000x reference)
import functools

import jax
import jax.numpy as jnp
from jax.experimental import pallas as pl
from jax.experimental.pallas import tpu as pltpu

B = 2
L = 2048
D_MODEL = 1024
HEADS = 16
DH = 64
K_TOP = 15
F = B * DH


def _proj_kernel(q_ref, k_ref, v_ref, wq_ref, bq_ref, bqc_ref,
                 qm_ref, km_ref, vt_ref):
    wq = wq_ref[...]
    qb = q_ref[0]
    kb = k_ref[0]
    vb = v_ref[0]
    qm_ref[...] = jnp.dot(qb, wq, preferred_element_type=jnp.float32,
                          precision=jax.lax.Precision.HIGHEST) + bq_ref[...]
    km_ref[...] = jnp.dot(kb, wq, preferred_element_type=jnp.float32,
                          precision=jax.lax.Precision.HIGHEST) + bq_ref[...]
    vt = jax.lax.dot_general(wq, vb, (((0,), (1,)), ((), ())),
                             preferred_element_type=jnp.float32,
                             precision=jax.lax.Precision.HIGHEST)
    vt_ref[...] = vt + bqc_ref[...]


def _corr_kernel(qm_ref, km_rev_ref, corr_ref):
    m = jax.lax.dot_general(qm_ref[...], km_rev_ref[...],
                            (((1,), (1,)), ((), ())),
                            preferred_element_type=jnp.float32,
                            precision=jax.lax.Precision.HIGHEST)
    a = m
    n = L
    while n > 8:
        h = n // 2
        a = a[:h, :] + pltpu.roll(a[h:, :], h, axis=1)
        n = h
    sub = jax.lax.broadcasted_iota(jnp.int32, (8, L), 0)
    for bit in (4, 2, 1):
        rolled = pltpu.roll(a, bit, axis=1)
        a = jnp.where((sub & bit) != 0, rolled, a)
    s = jnp.sum(a, axis=0, keepdims=True)
    corr_ref[...] = pltpu.roll(s, 1, axis=1) * (1.0 / F)


def _agg_kernel(corr_ref, vt_ref, r_ref):
    x = corr_ref[...]
    idx = jax.lax.broadcasted_iota(jnp.int32, (1, L), 1)
    neg = jnp.float32(-3.0e38)
    vals = []
    shifts = []
    for _ in range(K_TOP):
        mval = jnp.max(x)
        j = jnp.min(jnp.where(x == mval, idx, L))
        vals.append(mval)
        shifts.append((L - j) & (L - 1))
        x = jnp.where(idx == j, neg, x)
    exps = [jnp.exp(v - vals[0]) for v in vals]
    denom = functools.reduce(lambda p, q: p + q, exps)
    inv = 1.0 / denom
    vt = vt_ref[...]
    acc = jnp.zeros_like(vt)
    for i in range(K_TOP):
        w = exps[i] * inv
        acc = acc + w * pltpu.roll(vt, shifts[i], axis=1)
    r_ref[...] = acc


def _out_kernel(rs_ref, wo_ref, bo_ref, out_ref, d_ref):
    h = pl.program_id(1)

    @pl.when(h == 0)
    def _():
        d_ref[...] = jnp.dot(rs_ref[...], wo_ref[...],
                             preferred_element_type=jnp.float32,
                             precision=jax.lax.Precision.HIGHEST) + bo_ref[...]

    out_ref[0] = d_ref[...]


def kernel(Q, K, V, Wq, bq, Wo, bo):
    bq_row = bq.reshape(1, DH)
    bq_col = bq.reshape(DH, 1)
    bo_row = bo.reshape(1, D_MODEL)

    LT = 512
    nlt = L // LT
    qm, km, vt = pl.pallas_call(
        _proj_kernel,
        grid=(B, nlt),
        in_specs=[
            pl.BlockSpec((1, LT, D_MODEL), lambda b, lt: (b, lt, 0)),
            pl.BlockSpec((1, LT, D_MODEL), lambda b, lt: (b, lt, 0)),
            pl.BlockSpec((1, LT, D_MODEL), lambda b, lt: (b, lt, 0)),
            pl.BlockSpec((D_MODEL, DH), lambda b, lt: (0, 0)),
            pl.BlockSpec((1, DH), lambda b, lt: (0, 0)),
            pl.BlockSpec((DH, 1), lambda b, lt: (0, 0)),
        ],
        out_specs=[
            pl.BlockSpec((LT, DH), lambda b, lt: (lt, b)),
            pl.BlockSpec((LT, DH), lambda b, lt: (lt, b)),
            pl.BlockSpec((DH, LT), lambda b, lt: (b, lt)),
        ],
        out_shape=[
            jax.ShapeDtypeStruct((L, F), jnp.float32),
            jax.ShapeDtypeStruct((L, F), jnp.float32),
            jax.ShapeDtypeStruct((F, L), jnp.float32),
        ],
    )(Q, K, V, Wq, bq_row, bq_col)

    km_rev = km[::-1, :]

    corr = pl.pallas_call(
        _corr_kernel,
        in_specs=[
            pl.BlockSpec((L, F), lambda: (0, 0)),
            pl.BlockSpec((L, F), lambda: (0, 0)),
        ],
        out_specs=pl.BlockSpec((1, L), lambda: (0, 0)),
        out_shape=jax.ShapeDtypeStruct((1, L), jnp.float32),
        compiler_params=pltpu.CompilerParams(
            vmem_limit_bytes=60 * 1024 * 1024),
    )(qm, km_rev)

    r = pl.pallas_call(
        _agg_kernel,
        in_specs=[
            pl.BlockSpec((1, L), lambda: (0, 0)),
            pl.BlockSpec((F, L), lambda: (0, 0)),
        ],
        out_specs=pl.BlockSpec((F, L), lambda: (0, 0)),
        out_shape=jax.ShapeDtypeStruct((F, L), jnp.float32),
    )(corr, vt)

    rs = r.reshape(2 * F, D_MODEL)

    out = pl.pallas_call(
        _out_kernel,
        grid=(B, HEADS),
        in_specs=[
            pl.BlockSpec((2 * F // B, D_MODEL), lambda b, h: (b, 0)),
            pl.BlockSpec((D_MODEL, D_MODEL), lambda b, h: (0, 0)),
            pl.BlockSpec((1, D_MODEL), lambda b, h: (0, 0)),
        ],
        out_specs=pl.BlockSpec((1, 2 * F // B, D_MODEL),
                               lambda b, h: (b, h, 0)),
        out_shape=jax.ShapeDtypeStruct((B, L, D_MODEL), jnp.float32),
        scratch_shapes=[pltpu.VMEM((2 * F // B, D_MODEL), jnp.float32)],
    )(rs, Wo, bo_row)

    return out


# speedup vs baseline: 7.6338x; 7.6338x over previous
import functools

import jax
import jax.numpy as jnp
from jax.experimental import pallas as pl
from jax.experimental.pallas import tpu as pltpu

B = 2
L = 2048
D_MODEL = 1024
HEADS = 16
DH = 64
K_TOP = 15
F = B * DH


def _proj_kernel(q_ref, k_ref, v_ref, wq_ref, bq_ref, bqc_ref,
                 qm_ref, km_ref, vt_ref):
    wq = wq_ref[...]

    def proj(x):
        return jnp.dot(x, wq, preferred_element_type=jnp.float32,
                       precision=jax.lax.Precision.HIGHEST) + bq_ref[...]

    qm_ref[...] = jnp.concatenate([proj(q_ref[0]), proj(q_ref[1])], axis=1)
    km_ref[...] = jnp.concatenate([proj(k_ref[0]), proj(k_ref[1])], axis=1)

    def projt(x):
        vt = jax.lax.dot_general(wq, x, (((0,), (1,)), ((), ())),
                                 preferred_element_type=jnp.float32,
                                 precision=jax.lax.Precision.HIGHEST)
        return vt + bqc_ref[...]

    vt_ref[...] = jnp.concatenate([projt(v_ref[0]), projt(v_ref[1])], axis=0)


def _corr_kernel(qm_ref, km_rev_ref, corr_ref):
    m = jax.lax.dot_general(qm_ref[...], km_rev_ref[...],
                            (((1,), (1,)), ((), ())),
                            preferred_element_type=jnp.float32,
                            precision=jax.lax.Precision.HIGHEST)
    a = m
    n = L
    while n > 8:
        h = n // 2
        a = a[:h, :] + pltpu.roll(a[h:, :], h, axis=1)
        n = h
    sub = jax.lax.broadcasted_iota(jnp.int32, (8, L), 0)
    for bit in (4, 2, 1):
        rolled = pltpu.roll(a, bit, axis=1)
        a = jnp.where((sub & bit) != 0, rolled, a)
    s = jnp.sum(a, axis=0, keepdims=True)
    corr_ref[...] = pltpu.roll(s, 1, axis=1) * (1.0 / F)


def _agg_kernel(corr_ref, vt_ref, r_ref):
    x = corr_ref[...]
    idx = jax.lax.broadcasted_iota(jnp.int32, (1, L), 1)
    neg = jnp.float32(-3.0e38)
    vals = []
    shifts = []
    for _ in range(K_TOP):
        mval = jnp.max(x)
        j = jnp.min(jnp.where(x == mval, idx, L))
        vals.append(mval)
        shifts.append((L - j) & (L - 1))
        x = jnp.where(idx == j, neg, x)
    exps = [jnp.exp(v - vals[0]) for v in vals]
    denom = functools.reduce(lambda p, q: p + q, exps)
    inv = 1.0 / denom
    vt = vt_ref[...]
    acc = jnp.zeros_like(vt)
    for i in range(K_TOP):
        w = exps[i] * inv
        acc = acc + w * pltpu.roll(vt, shifts[i], axis=1)
    r_ref[...] = acc


def _out_kernel(rs_ref, wo_ref, bo_ref, out_ref, d_ref):
    h = pl.program_id(1)

    @pl.when(h == 0)
    def _():
        d_ref[...] = jnp.dot(rs_ref[...], wo_ref[...],
                             preferred_element_type=jnp.float32,
                             precision=jax.lax.Precision.HIGHEST) + bo_ref[...]

    out_ref[0] = d_ref[...]


def kernel(Q, K, V, Wq, bq, Wo, bo):
    bq_row = bq.reshape(1, DH)
    bq_col = bq.reshape(DH, 1)
    bo_row = bo.reshape(1, D_MODEL)

    LT = 512
    nlt = L // LT
    qm, km, vt = pl.pallas_call(
        _proj_kernel,
        grid=(nlt,),
        in_specs=[
            pl.BlockSpec((B, LT, D_MODEL), lambda lt: (0, lt, 0)),
            pl.BlockSpec((B, LT, D_MODEL), lambda lt: (0, lt, 0)),
            pl.BlockSpec((B, LT, D_MODEL), lambda lt: (0, lt, 0)),
            pl.BlockSpec((D_MODEL, DH), lambda lt: (0, 0)),
            pl.BlockSpec((1, DH), lambda lt: (0, 0)),
            pl.BlockSpec((DH, 1), lambda lt: (0, 0)),
        ],
        out_specs=[
            pl.BlockSpec((LT, F), lambda lt: (lt, 0)),
            pl.BlockSpec((LT, F), lambda lt: (lt, 0)),
            pl.BlockSpec((F, LT), lambda lt: (0, lt)),
        ],
        out_shape=[
            jax.ShapeDtypeStruct((L, F), jnp.float32),
            jax.ShapeDtypeStruct((L, F), jnp.float32),
            jax.ShapeDtypeStruct((F, L), jnp.float32),
        ],
    )(Q, K, V, Wq, bq_row, bq_col)

    km_rev = km[::-1, :]

    corr = pl.pallas_call(
        _corr_kernel,
        in_specs=[
            pl.BlockSpec((L, F), lambda: (0, 0)),
            pl.BlockSpec((L, F), lambda: (0, 0)),
        ],
        out_specs=pl.BlockSpec((1, L), lambda: (0, 0)),
        out_shape=jax.ShapeDtypeStruct((1, L), jnp.float32),
        compiler_params=pltpu.CompilerParams(
            vmem_limit_bytes=60 * 1024 * 1024),
    )(qm, km_rev)

    r = pl.pallas_call(
        _agg_kernel,
        in_specs=[
            pl.BlockSpec((1, L), lambda: (0, 0)),
            pl.BlockSpec((F, L), lambda: (0, 0)),
        ],
        out_specs=pl.BlockSpec((F, L), lambda: (0, 0)),
        out_shape=jax.ShapeDtypeStruct((F, L), jnp.float32),
    )(corr, vt)

    rs = r.reshape(2 * F, D_MODEL)

    out = pl.pallas_call(
        _out_kernel,
        grid=(B, HEADS),
        in_specs=[
            pl.BlockSpec((2 * F // B, D_MODEL), lambda b, h: (b, 0)),
            pl.BlockSpec((D_MODEL, D_MODEL), lambda b, h: (0, 0)),
            pl.BlockSpec((1, D_MODEL), lambda b, h: (0, 0)),
        ],
        out_specs=pl.BlockSpec((1, 2 * F // B, D_MODEL),
                               lambda b, h: (b, h, 0)),
        out_shape=jax.ShapeDtypeStruct((B, L, D_MODEL), jnp.float32),
        scratch_shapes=[pltpu.VMEM((2 * F // B, D_MODEL), jnp.float32)],
    )(rs, Wo, bo_row)

    return out


# speedup vs baseline: 12.8789x; 1.6871x over previous
import functools

import jax
import jax.numpy as jnp
from jax.experimental import pallas as pl
from jax.experimental.pallas import tpu as pltpu

B = 2
L = 2048
D_MODEL = 1024
HEADS = 16
DH = 64
K_TOP = 15
F = B * DH
_PREC = jax.lax.Precision.DEFAULT
_PREC_CORR = jax.lax.Precision.HIGHEST


def _proj_kernel(q_ref, k_ref, v_ref, wq_ref, bqc_ref,
                 qt_ref, kt_ref, vt_ref):
    wq = wq_ref[...]
    bqc = bqc_ref[...]

    def projt(x):
        y = jax.lax.dot_general(wq, x, (((0,), (1,)), ((), ())),
                                preferred_element_type=jnp.float32,
                                precision=_PREC)
        return y + bqc

    qt_ref[...] = jnp.concatenate([projt(q_ref[0]), projt(q_ref[1])], axis=0)
    kt_ref[...] = jnp.concatenate([projt(k_ref[0]), projt(k_ref[1])], axis=0)
    vt_ref[...] = jnp.concatenate([projt(v_ref[0]), projt(v_ref[1])], axis=0)


def _corr_agg_kernel(qt_ref, kt_ref, vt_ref, r_ref):
    m = jax.lax.dot_general(qt_ref[...], kt_ref[...],
                            (((0,), (0,)), ((), ())),
                            preferred_element_type=jnp.float32,
                            precision=_PREC_CORR)
    a = m
    n = L
    while n > 8:
        h = n // 2
        a = a[:h, :] + pltpu.roll(a[h:, :], L - h, axis=1)
        n = h
    sub = jax.lax.broadcasted_iota(jnp.int32, (8, L), 0)
    for bit in (4, 2, 1):
        rolled = pltpu.roll(a, L - bit, axis=1)
        a = jnp.where((sub & bit) != 0, rolled, a)
    x = jnp.sum(a, axis=0, keepdims=True) * (1.0 / F)
    idx = jax.lax.broadcasted_iota(jnp.int32, (1, L), 1)
    neg = jnp.float32(-3.0e38)
    vals = []
    shifts = []
    for _ in range(K_TOP):
        mval = jnp.max(x)
        j = jnp.min(jnp.where(x == mval, idx, L))
        vals.append(mval)
        shifts.append(j)
        x = jnp.where(idx == j, neg, x)
    exps = [jnp.exp(v - vals[0]) for v in vals]
    denom = functools.reduce(lambda p, q: p + q, exps)
    inv = 1.0 / denom

    vt = vt_ref[...]
    acc = jnp.zeros_like(vt)
    for i in range(K_TOP):
        acc = acc + (exps[i] * inv) * pltpu.roll(vt, shifts[i], axis=1)
    r_ref[...] = acc


def _out_kernel(rs_ref, wo_ref, bo_ref, out_ref, d_ref):
    h = pl.program_id(1)

    @pl.when(h == 0)
    def _():
        d_ref[...] = jnp.dot(rs_ref[...], wo_ref[...],
                             preferred_element_type=jnp.float32,
                             precision=_PREC) + bo_ref[...]

    out_ref[0] = d_ref[...]


def kernel(Q, K, V, Wq, bq, Wo, bo):
    bq_col = bq.reshape(DH, 1)
    bo_row = bo.reshape(1, D_MODEL)

    LT = 512
    nlt = L // LT
    qt, kt, vt = pl.pallas_call(
        _proj_kernel,
        grid=(nlt,),
        in_specs=[
            pl.BlockSpec((B, LT, D_MODEL), lambda lt: (0, lt, 0)),
            pl.BlockSpec((B, LT, D_MODEL), lambda lt: (0, lt, 0)),
            pl.BlockSpec((B, LT, D_MODEL), lambda lt: (0, lt, 0)),
            pl.BlockSpec((D_MODEL, DH), lambda lt: (0, 0)),
            pl.BlockSpec((DH, 1), lambda lt: (0, 0)),
        ],
        out_specs=[
            pl.BlockSpec((F, LT), lambda lt: (0, lt)),
            pl.BlockSpec((F, LT), lambda lt: (0, lt)),
            pl.BlockSpec((F, LT), lambda lt: (0, lt)),
        ],
        out_shape=[
            jax.ShapeDtypeStruct((F, L), jnp.float32),
            jax.ShapeDtypeStruct((F, L), jnp.float32),
            jax.ShapeDtypeStruct((F, L), jnp.float32),
        ],
    )(Q, K, V, Wq, bq_col)

    r = pl.pallas_call(
        _corr_agg_kernel,
        in_specs=[
            pl.BlockSpec((F, L), lambda: (0, 0)),
            pl.BlockSpec((F, L), lambda: (0, 0)),
            pl.BlockSpec((F, L), lambda: (0, 0)),
        ],
        out_specs=pl.BlockSpec((F, L), lambda: (0, 0)),
        out_shape=jax.ShapeDtypeStruct((F, L), jnp.float32),
        compiler_params=pltpu.CompilerParams(
            vmem_limit_bytes=60 * 1024 * 1024),
    )(qt, kt, vt)

    rs = r.reshape(2 * F, D_MODEL)

    out = pl.pallas_call(
        _out_kernel,
        grid=(B, HEADS),
        in_specs=[
            pl.BlockSpec((2 * F // B, D_MODEL), lambda b, h: (b, 0)),
            pl.BlockSpec((D_MODEL, D_MODEL), lambda b, h: (0, 0)),
            pl.BlockSpec((1, D_MODEL), lambda b, h: (0, 0)),
        ],
        out_specs=pl.BlockSpec((1, 2 * F // B, D_MODEL),
                               lambda b, h: (b, h, 0)),
        out_shape=jax.ShapeDtypeStruct((B, L, D_MODEL), jnp.float32),
        scratch_shapes=[pltpu.VMEM((2 * F // B, D_MODEL), jnp.float32)],
    )(rs, Wo, bo_row)

    return out


# speedup vs baseline: 17.7808x; 1.3806x over previous
import functools

import jax
import jax.numpy as jnp
from jax.experimental import pallas as pl
from jax.experimental.pallas import tpu as pltpu

B = 2
L = 2048
D_MODEL = 1024
HEADS = 16
DH = 64
K_TOP = 15
F = B * DH


def _proj_kernel(q_ref, k_ref, v_ref, wq_ref, bqc_ref,
                 qt_ref, kt_ref, vt_ref):
    wq = wq_ref[...]
    bqc = bqc_ref[...]

    def projt(x):
        y = jax.lax.dot_general(wq, x, (((0,), (1,)), ((), ())),
                                preferred_element_type=jnp.float32)
        return y + bqc

    qt_ref[...] = jnp.concatenate([projt(q_ref[0]), projt(q_ref[1])], axis=0)
    kt_ref[...] = jnp.concatenate([projt(k_ref[0]), projt(k_ref[1])], axis=0)
    vt_ref[...] = jnp.concatenate([projt(v_ref[0]), projt(v_ref[1])], axis=0)


def _dot_ff(a, b):
    return jax.lax.dot_general(a, b, (((0,), (0,)), ((), ())),
                               preferred_element_type=jnp.float32)


def _corr_out_kernel(qt_ref, kt_ref, vt_ref, wo_ref, bo_ref, out_ref, d_ref):
    b = pl.program_id(0)
    half = pl.program_id(1)

    @pl.when(jnp.logical_and(b == 0, half == 0))
    def _():
        qt = qt_ref[...]
        kt = kt_ref[...]
        qh = qt.astype(jnp.bfloat16)
        ql = (qt - qh.astype(jnp.float32)).astype(jnp.bfloat16)
        kh = kt.astype(jnp.bfloat16)
        kl = (kt - kh.astype(jnp.float32)).astype(jnp.bfloat16)
        m = _dot_ff(qh, kh) + _dot_ff(qh, kl) + _dot_ff(ql, kh)

        a = m
        n = L
        while n > 8:
            h = n // 2
            a = a[:h, :] + pltpu.roll(a[h:, :], L - h, axis=1)
            n = h
        sub = jax.lax.broadcasted_iota(jnp.int32, (8, L), 0)
        for bit in (4, 2, 1):
            rolled = pltpu.roll(a, L - bit, axis=1)
            a = jnp.where((sub & bit) != 0, rolled, a)
        x = jnp.sum(a, axis=0, keepdims=True) * (1.0 / F)
        idx = jax.lax.broadcasted_iota(jnp.int32, (1, L), 1)
        neg = jnp.float32(-3.0e38)
        vals = []
        shifts = []
        for _ in range(K_TOP):
            mval = jnp.max(x)
            j = jnp.min(jnp.where(x == mval, idx, L))
            vals.append(mval)
            shifts.append(j)
            x = jnp.where(idx == j, neg, x)
        exps = [jnp.exp(v - vals[0]) for v in vals]
        denom = functools.reduce(lambda p, q: p + q, exps)
        inv = 1.0 / denom

        vt = vt_ref[...]
        acc = jnp.zeros_like(vt)
        for i in range(K_TOP):
            acc = acc + (exps[i] * inv) * pltpu.roll(vt, shifts[i], axis=1)

        rows = jax.lax.broadcasted_iota(jnp.int32, (F, DH), 0)
        cols = jax.lax.broadcasted_iota(jnp.int32, (F, DH), 1)
        pe = (rows == 2 * cols).astype(jnp.float32)
        po = (rows == 2 * cols + 1).astype(jnp.float32)
        wo = wo_ref[...]
        bo = bo_ref[...]
        for bb in range(B):
            accb = acc[bb * DH:(bb + 1) * DH, :]
            rsb = (jnp.dot(pe, accb[:, :D_MODEL],
                           preferred_element_type=jnp.float32) +
                   jnp.dot(po, accb[:, D_MODEL:],
                           preferred_element_type=jnp.float32))
            d_ref[bb] = jnp.dot(rsb, wo,
                                preferred_element_type=jnp.float32) + bo

    d = d_ref[b]
    out_ref[0] = jnp.concatenate([d] * 8, axis=0)


def kernel(Q, K, V, Wq, bq, Wo, bo):
    bq_col = bq.reshape(DH, 1)
    bo_row = bo.reshape(1, D_MODEL)

    LT = 512
    nlt = L // LT
    qt, kt, vt = pl.pallas_call(
        _proj_kernel,
        grid=(nlt,),
        in_specs=[
            pl.BlockSpec((B, LT, D_MODEL), lambda lt: (0, lt, 0)),
            pl.BlockSpec((B, LT, D_MODEL), lambda lt: (0, lt, 0)),
            pl.BlockSpec((B, LT, D_MODEL), lambda lt: (0, lt, 0)),
            pl.BlockSpec((D_MODEL, DH), lambda lt: (0, 0)),
            pl.BlockSpec((DH, 1), lambda lt: (0, 0)),
        ],
        out_specs=[
            pl.BlockSpec((F, LT), lambda lt: (0, lt)),
            pl.BlockSpec((F, LT), lambda lt: (0, lt)),
            pl.BlockSpec((F, LT), lambda lt: (0, lt)),
        ],
        out_shape=[
            jax.ShapeDtypeStruct((F, L), jnp.float32),
            jax.ShapeDtypeStruct((F, L), jnp.float32),
            jax.ShapeDtypeStruct((F, L), jnp.float32),
        ],
    )(Q, K, V, Wq, bq_col)

    out = pl.pallas_call(
        _corr_out_kernel,
        grid=(B, 2),
        in_specs=[
            pl.BlockSpec((F, L), lambda b, h: (0, 0)),
            pl.BlockSpec((F, L), lambda b, h: (0, 0)),
            pl.BlockSpec((F, L), lambda b, h: (0, 0)),
            pl.BlockSpec((D_MODEL, D_MODEL), lambda b, h: (0, 0)),
            pl.BlockSpec((1, D_MODEL), lambda b, h: (0, 0)),
        ],
        out_specs=pl.BlockSpec((1, L // 2, D_MODEL), lambda b, h: (b, h, 0)),
        out_shape=jax.ShapeDtypeStruct((B, L, D_MODEL), jnp.float32),
        scratch_shapes=[pltpu.VMEM((B, F, D_MODEL), jnp.float32)],
        compiler_params=pltpu.CompilerParams(
            vmem_limit_bytes=60 * 1024 * 1024),
    )(qt, kt, vt, Wo, bo_row)

    return out


# speedup vs baseline: 18.8686x; 1.0612x over previous
import functools

import jax
import jax.numpy as jnp
from jax.experimental import pallas as pl
from jax.experimental.pallas import tpu as pltpu

B = 2
L = 2048
D_MODEL = 1024
HEADS = 16
DH = 64
K_TOP = 15
F = B * DH
LT = 256
NLT = L // LT


def _dot_ff(a, b):
    return jax.lax.dot_general(a, b, (((0,), (0,)), ((), ())),
                               preferred_element_type=jnp.float32)


def _bf16_split(x):
    hi = x.astype(jnp.bfloat16)
    lo = (x - hi.astype(jnp.float32)).astype(jnp.bfloat16)
    return hi, lo


def _fused_kernel(q_ref, k_ref, v_ref, wq_ref, bqc_ref, wo_ref, bo_ref,
                  out_ref, qs_ref, ks_ref, vs_ref, d_ref):
    s = pl.program_id(0)

    @pl.when(s < NLT)
    def _proj():
        wq = wq_ref[...]
        bqc = bqc_ref[...]

        def projt(x):
            y = jax.lax.dot_general(wq, x, (((0,), (1,)), ((), ())),
                                    preferred_element_type=jnp.float32)
            return y + bqc

        qs_ref[s] = jnp.concatenate([projt(q_ref[0]), projt(q_ref[1])], 0)
        ks_ref[s] = jnp.concatenate([projt(k_ref[0]), projt(k_ref[1])], 0)
        vs_ref[s] = jnp.concatenate([projt(v_ref[0]), projt(v_ref[1])], 0)

    @pl.when(s == NLT)
    def _corr_agg():
        qt = jnp.concatenate([qs_ref[i] for i in range(NLT)], axis=1)
        kt = jnp.concatenate([ks_ref[i] for i in range(NLT)], axis=1)
        qh, ql = _bf16_split(qt)
        kh, kl = _bf16_split(kt)

        a = None
        for q4 in range(4):
            sl = slice(512 * q4, 512 * (q4 + 1))
            mq = (_dot_ff(qh[:, sl], kh) + _dot_ff(qh[:, sl], kl) +
                  _dot_ff(ql[:, sl], kh))
            if q4:
                a = a + pltpu.roll(mq, (L - 512 * q4) % L, axis=1)
            else:
                a = mq
        n = 512
        while n > 8:
            h = n // 2
            a = a[:h, :] + pltpu.roll(a[h:, :], L - h, axis=1)
            n = h
        sub = jax.lax.broadcasted_iota(jnp.int32, (8, L), 0)
        for bit in (4, 2, 1):
            rolled = pltpu.roll(a, L - bit, axis=1)
            a = jnp.where((sub & bit) != 0, rolled, a)
        x = jnp.sum(a, axis=0, keepdims=True) * (1.0 / F)
        idx = jax.lax.broadcasted_iota(jnp.int32, (1, L), 1)
        neg = jnp.float32(-3.0e38)
        vals = []
        shifts = []
        for _ in range(K_TOP):
            mval = jnp.max(x)
            j = jnp.min(jnp.where(x == mval, idx, L))
            vals.append(mval)
            shifts.append(j)
            x = jnp.where(idx == j, neg, x)
        exps = [jnp.exp(v - vals[0]) for v in vals]
        denom = functools.reduce(lambda p, q: p + q, exps)
        inv = 1.0 / denom

        vt = jnp.concatenate([vs_ref[i] for i in range(NLT)], axis=1)
        acc = jnp.zeros_like(vt)
        for i in range(K_TOP):
            acc = acc + (exps[i] * inv) * pltpu.roll(vt, shifts[i], axis=1)

        rows = jax.lax.broadcasted_iota(jnp.int32, (F, DH), 0)
        cols = jax.lax.broadcasted_iota(jnp.int32, (F, DH), 1)
        pe = (rows == 2 * cols).astype(jnp.float32)
        po = (rows == 2 * cols + 1).astype(jnp.float32)
        wo = wo_ref[...]
        bo = bo_ref[...]
        for bb in range(B):
            accb = acc[bb * DH:(bb + 1) * DH, :]
            rsb = (jnp.dot(pe, accb[:, :D_MODEL],
                           preferred_element_type=jnp.float32) +
                   jnp.dot(po, accb[:, D_MODEL:],
                           preferred_element_type=jnp.float32))
            d_ref[bb] = jnp.dot(rsb, wo,
                                preferred_element_type=jnp.float32) + bo

    @pl.when(s >= NLT)
    def _write():
        b = (s - NLT) // 2
        d = d_ref[b]
        out_ref[0] = jnp.concatenate([d] * 8, axis=0)


def kernel(Q, K, V, Wq, bq, Wo, bo):
    bq_col = bq.reshape(DH, 1)
    bo_row = bo.reshape(1, D_MODEL)

    def in_idx(s):
        t = jnp.minimum(s, NLT - 1)
        return (0, t, 0)

    def out_idx(s):
        t = jnp.maximum(s - NLT, 0)
        return (t // 2, t % 2, 0)

    out = pl.pallas_call(
        _fused_kernel,
        grid=(NLT + 4,),
        in_specs=[
            pl.BlockSpec((B, LT, D_MODEL), in_idx),
            pl.BlockSpec((B, LT, D_MODEL), in_idx),
            pl.BlockSpec((B, LT, D_MODEL), in_idx),
            pl.BlockSpec((D_MODEL, DH), lambda s: (0, 0)),
            pl.BlockSpec((DH, 1), lambda s: (0, 0)),
            pl.BlockSpec((D_MODEL, D_MODEL), lambda s: (0, 0)),
            pl.BlockSpec((1, D_MODEL), lambda s: (0, 0)),
        ],
        out_specs=pl.BlockSpec((1, L // 2, D_MODEL), out_idx),
        out_shape=jax.ShapeDtypeStruct((B, L, D_MODEL), jnp.float32),
        scratch_shapes=[
            pltpu.VMEM((NLT, F, LT), jnp.float32),
            pltpu.VMEM((NLT, F, LT), jnp.float32),
            pltpu.VMEM((NLT, F, LT), jnp.float32),
            pltpu.VMEM((B, F, D_MODEL), jnp.float32),
        ],
        compiler_params=pltpu.CompilerParams(
            vmem_limit_bytes=60 * 1024 * 1024),
    )(Q, K, V, Wq, bq_col, Wo, bo_row)

    return out
